# Initial kernel scaffold; baseline (speedup 1.0000x reference)
import jax
import jax.numpy as jnp
from jax.experimental import pallas as pl


def kernel(char_prem_batch, char_hypo_batch, char_prem_masks, char_hypo_masks, table):
    raise NotImplementedError("write your pallas kernel here")


# speedup vs baseline: 2.8670x; 2.8670x over previous
import functools

import jax
import jax.numpy as jnp
from jax import lax
from jax.experimental import pallas as pl
from jax.experimental.pallas import tpu as pltpu
from jax.experimental.pallas import tpu_sc as plsc

S, W, B = 128, 16, 256
V, D = 1000, 64
L = 16

_info = plsc.get_sparse_core_info()
NC, NS = _info.num_cores, _info.num_subcores
NW = NC * NS
S_PER_W = S // NW


def _sc_body(cp_hbm, ch_hbm, mp_hbm, mh_hbm, tab_hbm,
             out_p_hbm, out_h_hbm,
             table_v, idx_v, mask_v, out_v):
    cid = lax.axis_index("c")
    sid = lax.axis_index("s")
    wid = sid * NC + cid

    pltpu.sync_copy(tab_hbm, table_v)

    iota = lax.iota(jnp.int32, L)

    for idx_hbm, msk_hbm, out_hbm in ((cp_hbm, mp_hbm, out_p_hbm),
                                      (ch_hbm, mh_hbm, out_h_hbm)):
        def unit(j, carry, idx_hbm=idx_hbm, msk_hbm=msk_hbm, out_hbm=out_hbm):
            s = wid * S_PER_W + j
            pltpu.sync_copy(idx_hbm.at[s], idx_v)
            pltpu.sync_copy(msk_hbm.at[s], mask_v)

            for h in range(2):
                def bblk(blk, carry2, h=h):
                    b0 = h * (B // 2) + blk * L
                    base = []
                    mk = []
                    for w in range(W):
                        base.append(idx_v[w, pl.ds(b0, L)])
                        mk.append(mask_v[w, pl.ds(b0, L)] * (1.0 / W))
                    rows = iota + blk * L

                    def dloop(dq, carry3):
                        for dd in range(4):
                            d = dq * 4 + dd
                            dsp = lax.broadcast(d, (L,))
                            acc = mk[0] * plsc.load_gather(
                                table_v, [base[0], dsp])
                            for w in range(1, W):
                                acc = acc + mk[w] * plsc.load_gather(
                                    table_v, [base[w], dsp])
                            plsc.store_scatter(out_v, [rows, dsp], acc)
                        return carry3

                    lax.fori_loop(0, D // 4, dloop, 0)
                    return carry2

                lax.fori_loop(0, B // (2 * L), bblk, 0)
                pltpu.sync_copy(out_v,
                                out_hbm.at[s, pl.ds(h * (B // 2), B // 2)])
            return carry

        lax.fori_loop(0, S_PER_W, unit, 0)


@functools.partial(jax.jit, static_argnums=())
def _encode(cp, ch, mp, mh, tab_flat):
    mesh = plsc.VectorSubcoreMesh(core_axis_name="c", subcore_axis_name="s")
    f = pl.kernel(
        _sc_body,
        out_type=(
            jax.ShapeDtypeStruct((S, B, D), jnp.float32),
            jax.ShapeDtypeStruct((S, B, D), jnp.float32),
        ),
        mesh=mesh,
        compiler_params=pltpu.CompilerParams(
            needs_layout_passes=False, use_tc_tiling_on_sc=False),
        scratch_types=[
            pltpu.VMEM((V, D), jnp.float32),
            pltpu.VMEM((W, B), jnp.int32),
            pltpu.VMEM((W, B), jnp.float32),
            pltpu.VMEM((B // 2, D), jnp.float32),
        ],
    )
    return f(cp, ch, mp, mh, tab_flat)


def kernel(char_prem_batch, char_hypo_batch, char_prem_masks,
           char_hypo_masks, table):
    cp = char_prem_batch.astype(jnp.int32)
    ch = char_hypo_batch.astype(jnp.int32)
    return _encode(cp, ch, char_prem_masks, char_hypo_masks, table)
